# baseline (speedup 1.0000x reference)
import functools

import jax
import jax.numpy as jnp
from jax import lax
from jax.experimental import pallas as pl
from jax.experimental.pallas import tpu as pltpu
from jax.experimental.pallas import tpu_sc as plsc

B, N, D, D3 = 1024, 200, 128, 384
_NC, _NS, _L = 2, 16, 16
NW = _NC * _NS
BPW = B // NW
RN = 40
NDMA = N // RN
NCHUNK = D // _L


def _sc_body(idx_c_hbm, idx_a_hbm, idx_t_hbm,
             emb_c_hbm, emb_a_hbm, emb_t_hbm, out_hbm,
             idx_c_v, idx_a_v, idx_t_v, rows_c, rows_a, rows_t,
             buf0, buf1, gsem, osem):
    wid = lax.axis_index("s") * _NC + lax.axis_index("c")
    base = wid * BPW

    pltpu.sync_copy(idx_c_hbm.at[pl.ds(base, BPW)], idx_c_v)
    pltpu.sync_copy(idx_a_hbm.at[pl.ds(base, BPW)], idx_a_v)
    pltpu.sync_copy(idx_t_hbm.at[pl.ds(base, BPW)], idx_t_v)

    c_c = pltpu.async_copy(emb_c_hbm.at[idx_c_v], rows_c, gsem)
    c_a = pltpu.async_copy(emb_a_hbm.at[idx_a_v], rows_a, gsem)
    c_t = pltpu.async_copy(emb_t_hbm.at[idx_t_v], rows_t, gsem)
    c_c.wait()
    c_a.wait()
    c_t.wait()

    def _fill(buf, s):
        for k3, src in enumerate((rows_c, rows_a, rows_t)):
            for q in range(NCHUNK):
                v = src[s, pl.ds(q * _L, _L)]
                for r in range(RN):
                    buf[0, r, pl.ds(k3 * D + q * _L, _L)] = v

    def _issue(buf, s):
        b_abs = base + s
        for r8 in range(NDMA):
            pltpu.async_copy(
                buf, out_hbm.at[pl.ds(b_abs, 1), pl.ds(r8 * RN, RN), :], osem)

    def _drain(buf):
        for _ in range(NDMA):
            pltpu.make_async_copy(
                buf, out_hbm.at[pl.ds(base, 1), pl.ds(0, RN), :], osem).wait()

    def _pair(g, carry):
        @pl.when(g > 0)
        def _():
            _drain(buf0)
            _drain(buf1)
        _fill(buf0, 2 * g)
        _issue(buf0, 2 * g)
        _fill(buf1, 2 * g + 1)
        _issue(buf1, 2 * g + 1)
        return carry

    lax.fori_loop(0, BPW // 2, _pair, 0)
    _drain(buf0)
    _drain(buf1)


def kernel(x_t, idx_C, idx_A, idx_T, emb_C, emb_A, emb_T):
    mesh = plsc.VectorSubcoreMesh(core_axis_name="c", subcore_axis_name="s",
                                  num_cores=_NC, num_subcores=_NS)
    run = functools.partial(
        pl.kernel,
        mesh=mesh,
        out_type=jax.ShapeDtypeStruct((B, N, D3), jnp.float32),
        scratch_types=[
            pltpu.VMEM((BPW,), jnp.int32),
            pltpu.VMEM((BPW,), jnp.int32),
            pltpu.VMEM((BPW,), jnp.int32),
            pltpu.VMEM((BPW, D), jnp.float32),
            pltpu.VMEM((BPW, D), jnp.float32),
            pltpu.VMEM((BPW, D), jnp.float32),
            pltpu.VMEM((1, RN, D3), jnp.float32),
            pltpu.VMEM((1, RN, D3), jnp.float32),
            pltpu.SemaphoreType.DMA,
            pltpu.SemaphoreType.DMA,
        ],
    )(_sc_body)
    return run(idx_C.astype(jnp.int32), idx_A.astype(jnp.int32),
               idx_T.astype(jnp.int32), emb_C, emb_A, emb_T)


# speedup vs baseline: 1.1666x; 1.1666x over previous
import functools

import jax
import jax.numpy as jnp
from jax import lax
from jax.experimental import pallas as pl
from jax.experimental.pallas import tpu as pltpu
from jax.experimental.pallas import tpu_sc as plsc

B, N, D, D3 = 1024, 200, 128, 384
H1 = 512
H2 = B - H1
_NC, _NS, _L = 2, 16, 16
NW = _NC * _NS
BPW2 = H2 // NW
BB = 16


def _sc_gather_body(idx_c_hbm, idx_a_hbm, idx_t_hbm,
                    emb_c_hbm, emb_a_hbm, emb_t_hbm, fe_hbm,
                    idx_c_v, idx_a_v, idx_t_v, rows_c, rows_a, rows_t,
                    gsem, osem):
    wid = lax.axis_index("s") * _NC + lax.axis_index("c")
    base = H1 + wid * BPW2

    pltpu.sync_copy(idx_c_hbm.at[pl.ds(base, BPW2)], idx_c_v)
    pltpu.sync_copy(idx_a_hbm.at[pl.ds(base, BPW2)], idx_a_v)
    pltpu.sync_copy(idx_t_hbm.at[pl.ds(base, BPW2)], idx_t_v)

    c_c = pltpu.async_copy(emb_c_hbm.at[idx_c_v], rows_c, gsem)
    c_a = pltpu.async_copy(emb_a_hbm.at[idx_a_v], rows_a, gsem)
    c_t = pltpu.async_copy(emb_t_hbm.at[idx_t_v], rows_t, gsem)
    c_c.wait()
    c_a.wait()
    c_t.wait()

    out_base = wid * BPW2
    w_c = pltpu.async_copy(rows_c, fe_hbm.at[pl.ds(out_base, BPW2), pl.ds(0, D)], osem)
    w_a = pltpu.async_copy(rows_a, fe_hbm.at[pl.ds(out_base, BPW2), pl.ds(D, D)], osem)
    w_t = pltpu.async_copy(rows_t, fe_hbm.at[pl.ds(out_base, BPW2), pl.ds(2 * D, D)], osem)
    w_c.wait()
    w_a.wait()
    w_t.wait()


def _sc_gather(idx_C, idx_A, idx_T, emb_C, emb_A, emb_T):
    mesh = plsc.VectorSubcoreMesh(core_axis_name="c", subcore_axis_name="s",
                                  num_cores=_NC, num_subcores=_NS)
    run = functools.partial(
        pl.kernel,
        mesh=mesh,
        out_type=jax.ShapeDtypeStruct((H2, D3), jnp.float32),
        scratch_types=[
            pltpu.VMEM((BPW2,), jnp.int32),
            pltpu.VMEM((BPW2,), jnp.int32),
            pltpu.VMEM((BPW2,), jnp.int32),
            pltpu.VMEM((BPW2, D), jnp.float32),
            pltpu.VMEM((BPW2, D), jnp.float32),
            pltpu.VMEM((BPW2, D), jnp.float32),
            pltpu.SemaphoreType.DMA,
            pltpu.SemaphoreType.DMA,
        ],
    )(_sc_gather_body)
    return run(idx_C, idx_A, idx_T, emb_C, emb_A, emb_T)


def _tc1_kernel(idx_c_ref, idx_a_ref, idx_t_ref,
                emb_c_ref, emb_a_ref, emb_t_ref, out_ref):
    i = pl.program_id(0)
    b0 = i * BB
    for j in range(BB):
        c = idx_c_ref[b0 + j]
        a = idx_a_ref[b0 + j]
        t = idx_t_ref[b0 + j]
        row = jnp.concatenate([
            emb_c_ref[pl.ds(c, 1), :],
            emb_a_ref[pl.ds(a, 1), :],
            emb_t_ref[pl.ds(t, 1), :],
        ], axis=-1)
        out_ref[j, :, :] = jnp.broadcast_to(row, (N, D3))


def _tc_stage1(idx_C, idx_A, idx_T, emb_C, emb_A, emb_T):
    return pl.pallas_call(
        _tc1_kernel,
        grid_spec=pltpu.PrefetchScalarGridSpec(
            num_scalar_prefetch=3,
            grid=(H1 // BB,),
            in_specs=[
                pl.BlockSpec(emb_C.shape, lambda i, *_: (0, 0)),
                pl.BlockSpec(emb_A.shape, lambda i, *_: (0, 0)),
                pl.BlockSpec(emb_T.shape, lambda i, *_: (0, 0)),
            ],
            out_specs=pl.BlockSpec((BB, N, D3), lambda i, *_: (i, 0, 0)),
        ),
        out_shape=jax.ShapeDtypeStruct((B, N, D3), jnp.float32),
    )(idx_C, idx_A, idx_T, emb_C, emb_A, emb_T)


def _tc2_kernel(fe_ref, _aliased_ref, out_ref):
    out_ref[...] = jnp.broadcast_to(fe_ref[:, :, :], (BB, N, D3))


def _tc_stage2(fold_emb2, out1):
    return pl.pallas_call(
        _tc2_kernel,
        grid=(H2 // BB,),
        in_specs=[
            pl.BlockSpec((BB, 1, D3), lambda i: (i, 0, 0)),
            pl.BlockSpec(memory_space=pltpu.MemorySpace.HBM),
        ],
        out_specs=pl.BlockSpec((BB, N, D3), lambda i: (i + H1 // BB, 0, 0)),
        out_shape=jax.ShapeDtypeStruct((B, N, D3), jnp.float32),
        input_output_aliases={1: 0},
    )(fold_emb2, out1)


def kernel(x_t, idx_C, idx_A, idx_T, emb_C, emb_A, emb_T):
    ic = idx_C.astype(jnp.int32)
    ia = idx_A.astype(jnp.int32)
    it = idx_T.astype(jnp.int32)
    fe2 = _sc_gather(ic, ia, it, emb_C, emb_A, emb_T)
    out1 = _tc_stage1(ic, ia, it, emb_C, emb_A, emb_T)
    return _tc_stage2(fe2.reshape(H2, 1, D3), out1)


# speedup vs baseline: 1.2085x; 1.0359x over previous
import functools

import jax
import jax.numpy as jnp
from jax import lax
from jax.experimental import pallas as pl
from jax.experimental.pallas import tpu as pltpu
from jax.experimental.pallas import tpu_sc as plsc

B, N, D, D3 = 1024, 200, 128, 384
H1 = 768
H2 = B - H1
_NC, _NS, _L = 2, 16, 16
NW = _NC * _NS
BPW2 = H2 // NW
BB = 16


def _sc_gather_body(idx_c_hbm, idx_a_hbm, idx_t_hbm,
                    emb_c_hbm, emb_a_hbm, emb_t_hbm, fe_hbm,
                    idx_c_v, idx_a_v, idx_t_v, rows_c, rows_a, rows_t,
                    gsem, osem):
    wid = lax.axis_index("s") * _NC + lax.axis_index("c")
    base = H1 + wid * BPW2

    pltpu.sync_copy(idx_c_hbm.at[pl.ds(base, BPW2)], idx_c_v)
    pltpu.sync_copy(idx_a_hbm.at[pl.ds(base, BPW2)], idx_a_v)
    pltpu.sync_copy(idx_t_hbm.at[pl.ds(base, BPW2)], idx_t_v)

    c_c = pltpu.async_copy(emb_c_hbm.at[idx_c_v], rows_c, gsem)
    c_a = pltpu.async_copy(emb_a_hbm.at[idx_a_v], rows_a, gsem)
    c_t = pltpu.async_copy(emb_t_hbm.at[idx_t_v], rows_t, gsem)
    c_c.wait()
    c_a.wait()
    c_t.wait()

    out_base = wid * BPW2
    w_c = pltpu.async_copy(rows_c, fe_hbm.at[pl.ds(out_base, BPW2), pl.ds(0, D)], osem)
    w_a = pltpu.async_copy(rows_a, fe_hbm.at[pl.ds(out_base, BPW2), pl.ds(D, D)], osem)
    w_t = pltpu.async_copy(rows_t, fe_hbm.at[pl.ds(out_base, BPW2), pl.ds(2 * D, D)], osem)
    w_c.wait()
    w_a.wait()
    w_t.wait()


def _sc_gather(idx_C, idx_A, idx_T, emb_C, emb_A, emb_T):
    mesh = plsc.VectorSubcoreMesh(core_axis_name="c", subcore_axis_name="s",
                                  num_cores=_NC, num_subcores=_NS)
    run = functools.partial(
        pl.kernel,
        mesh=mesh,
        out_type=jax.ShapeDtypeStruct((H2, D3), jnp.float32),
        scratch_types=[
            pltpu.VMEM((BPW2,), jnp.int32),
            pltpu.VMEM((BPW2,), jnp.int32),
            pltpu.VMEM((BPW2,), jnp.int32),
            pltpu.VMEM((BPW2, D), jnp.float32),
            pltpu.VMEM((BPW2, D), jnp.float32),
            pltpu.VMEM((BPW2, D), jnp.float32),
            pltpu.SemaphoreType.DMA,
            pltpu.SemaphoreType.DMA,
        ],
    )(_sc_gather_body)
    return run(idx_C, idx_A, idx_T, emb_C, emb_A, emb_T)


def _tc1_kernel(idx_c_ref, idx_a_ref, idx_t_ref,
                emb_c_ref, emb_a_ref, emb_t_ref, out_ref):
    i = pl.program_id(0)
    b0 = i * BB
    for j in range(BB):
        c = idx_c_ref[b0 + j]
        a = idx_a_ref[b0 + j]
        t = idx_t_ref[b0 + j]
        row = jnp.concatenate([
            emb_c_ref[pl.ds(c, 1), :],
            emb_a_ref[pl.ds(a, 1), :],
            emb_t_ref[pl.ds(t, 1), :],
        ], axis=-1)
        out_ref[j, :, :] = jnp.broadcast_to(row, (N, D3))


def _tc_stage1(idx_C, idx_A, idx_T, emb_C, emb_A, emb_T):
    return pl.pallas_call(
        _tc1_kernel,
        grid_spec=pltpu.PrefetchScalarGridSpec(
            num_scalar_prefetch=3,
            grid=(H1 // BB,),
            in_specs=[
                pl.BlockSpec(emb_C.shape, lambda i, *_: (0, 0)),
                pl.BlockSpec(emb_A.shape, lambda i, *_: (0, 0)),
                pl.BlockSpec(emb_T.shape, lambda i, *_: (0, 0)),
            ],
            out_specs=pl.BlockSpec((BB, N, D3), lambda i, *_: (i, 0, 0)),
        ),
        out_shape=jax.ShapeDtypeStruct((B, N, D3), jnp.float32),
    )(idx_C, idx_A, idx_T, emb_C, emb_A, emb_T)


def _tc2_kernel(fe_ref, _aliased_ref, out_ref):
    out_ref[...] = jnp.broadcast_to(fe_ref[:, :, :], (BB, N, D3))


def _tc_stage2(fold_emb2, out1):
    return pl.pallas_call(
        _tc2_kernel,
        grid=(H2 // BB,),
        in_specs=[
            pl.BlockSpec((BB, 1, D3), lambda i: (i, 0, 0)),
            pl.BlockSpec(memory_space=pltpu.MemorySpace.HBM),
        ],
        out_specs=pl.BlockSpec((BB, N, D3), lambda i: (i + H1 // BB, 0, 0)),
        out_shape=jax.ShapeDtypeStruct((B, N, D3), jnp.float32),
        input_output_aliases={1: 0},
    )(fold_emb2, out1)


def kernel(x_t, idx_C, idx_A, idx_T, emb_C, emb_A, emb_T):
    ic = idx_C.astype(jnp.int32)
    ia = idx_A.astype(jnp.int32)
    it = idx_T.astype(jnp.int32)
    fe2 = _sc_gather(ic, ia, it, emb_C, emb_A, emb_T)
    out1 = _tc_stage1(ic, ia, it, emb_C, emb_A, emb_T)
    return _tc_stage2(fe2.reshape(H2, 1, D3), out1)


# speedup vs baseline: 1.2173x; 1.0073x over previous
import functools

import jax
import jax.numpy as jnp
from jax import lax
from jax.experimental import pallas as pl
from jax.experimental.pallas import tpu as pltpu
from jax.experimental.pallas import tpu_sc as plsc

B, N, D, D3 = 1024, 200, 128, 384
H1 = 768
H2 = B - H1
_NC, _NS, _L = 2, 16, 16
NW = _NC * _NS
BPW2 = H2 // NW
BB = 32


def _sc_gather_body(idx_c_hbm, idx_a_hbm, idx_t_hbm,
                    emb_c_hbm, emb_a_hbm, emb_t_hbm, fe_hbm,
                    idx_c_v, idx_a_v, idx_t_v, rows_c, rows_a, rows_t,
                    gsem, osem):
    wid = lax.axis_index("s") * _NC + lax.axis_index("c")
    base = H1 + wid * BPW2

    pltpu.sync_copy(idx_c_hbm.at[pl.ds(base, BPW2)], idx_c_v)
    pltpu.sync_copy(idx_a_hbm.at[pl.ds(base, BPW2)], idx_a_v)
    pltpu.sync_copy(idx_t_hbm.at[pl.ds(base, BPW2)], idx_t_v)

    c_c = pltpu.async_copy(emb_c_hbm.at[idx_c_v], rows_c, gsem)
    c_a = pltpu.async_copy(emb_a_hbm.at[idx_a_v], rows_a, gsem)
    c_t = pltpu.async_copy(emb_t_hbm.at[idx_t_v], rows_t, gsem)
    c_c.wait()
    c_a.wait()
    c_t.wait()

    out_base = wid * BPW2
    w_c = pltpu.async_copy(rows_c, fe_hbm.at[pl.ds(out_base, BPW2), pl.ds(0, D)], osem)
    w_a = pltpu.async_copy(rows_a, fe_hbm.at[pl.ds(out_base, BPW2), pl.ds(D, D)], osem)
    w_t = pltpu.async_copy(rows_t, fe_hbm.at[pl.ds(out_base, BPW2), pl.ds(2 * D, D)], osem)
    w_c.wait()
    w_a.wait()
    w_t.wait()


def _sc_gather(idx_C, idx_A, idx_T, emb_C, emb_A, emb_T):
    mesh = plsc.VectorSubcoreMesh(core_axis_name="c", subcore_axis_name="s",
                                  num_cores=_NC, num_subcores=_NS)
    run = functools.partial(
        pl.kernel,
        mesh=mesh,
        out_type=jax.ShapeDtypeStruct((H2, D3), jnp.float32),
        scratch_types=[
            pltpu.VMEM((BPW2,), jnp.int32),
            pltpu.VMEM((BPW2,), jnp.int32),
            pltpu.VMEM((BPW2,), jnp.int32),
            pltpu.VMEM((BPW2, D), jnp.float32),
            pltpu.VMEM((BPW2, D), jnp.float32),
            pltpu.VMEM((BPW2, D), jnp.float32),
            pltpu.SemaphoreType.DMA,
            pltpu.SemaphoreType.DMA,
        ],
    )(_sc_gather_body)
    return run(idx_C, idx_A, idx_T, emb_C, emb_A, emb_T)


def _tc1_kernel(idx_c_ref, idx_a_ref, idx_t_ref,
                emb_c_ref, emb_a_ref, emb_t_ref, out_ref):
    i = pl.program_id(0)
    b0 = i * BB
    for j in range(BB):
        c = idx_c_ref[b0 + j]
        a = idx_a_ref[b0 + j]
        t = idx_t_ref[b0 + j]
        row = jnp.concatenate([
            emb_c_ref[pl.ds(c, 1), :],
            emb_a_ref[pl.ds(a, 1), :],
            emb_t_ref[pl.ds(t, 1), :],
        ], axis=-1)
        out_ref[j, :, :] = jnp.broadcast_to(row, (N, D3))


def _tc_stage1(idx_C, idx_A, idx_T, emb_C, emb_A, emb_T):
    return pl.pallas_call(
        _tc1_kernel,
        grid_spec=pltpu.PrefetchScalarGridSpec(
            num_scalar_prefetch=3,
            grid=(H1 // BB,),
            in_specs=[
                pl.BlockSpec(emb_C.shape, lambda i, *_: (0, 0)),
                pl.BlockSpec(emb_A.shape, lambda i, *_: (0, 0)),
                pl.BlockSpec(emb_T.shape, lambda i, *_: (0, 0)),
            ],
            out_specs=pl.BlockSpec((BB, N, D3), lambda i, *_: (i, 0, 0)),
        ),
        out_shape=jax.ShapeDtypeStruct((B, N, D3), jnp.float32),
    )(idx_C, idx_A, idx_T, emb_C, emb_A, emb_T)


def _tc2_kernel(fe_ref, _aliased_ref, out_ref):
    fe = fe_ref[...].reshape(BB, 1, D3)
    out_ref[...] = jnp.broadcast_to(fe, (BB, N, D3))


def _tc_stage2(fold_emb2, out1):
    return pl.pallas_call(
        _tc2_kernel,
        grid=(H2 // BB,),
        in_specs=[
            pl.BlockSpec((BB, D3), lambda i: (i, 0)),
            pl.BlockSpec(memory_space=pltpu.MemorySpace.HBM),
        ],
        out_specs=pl.BlockSpec((BB, N, D3), lambda i: (i + H1 // BB, 0, 0)),
        out_shape=jax.ShapeDtypeStruct((B, N, D3), jnp.float32),
        input_output_aliases={1: 0},
    )(fold_emb2, out1)


def kernel(x_t, idx_C, idx_A, idx_T, emb_C, emb_A, emb_T):
    ic = idx_C.astype(jnp.int32)
    ia = idx_A.astype(jnp.int32)
    it = idx_T.astype(jnp.int32)
    fe2 = _sc_gather(ic, ia, it, emb_C, emb_A, emb_T)
    out1 = _tc_stage1(ic, ia, it, emb_C, emb_A, emb_T)
    return _tc_stage2(fe2, out1)


# speedup vs baseline: 1.2301x; 1.0104x over previous
import functools

import jax
import jax.numpy as jnp
from jax import lax
from jax.experimental import pallas as pl
from jax.experimental.pallas import tpu as pltpu
from jax.experimental.pallas import tpu_sc as plsc

B, N, D, D3 = 1024, 200, 128, 384
H1 = 768
H2 = B - H1
_NC, _NS, _L = 2, 16, 16
NW = _NC * _NS
BPW2 = H2 // NW
BB = 32


def _sc_gather_body(idx_c_hbm, idx_a_hbm, idx_t_hbm,
                    emb_c_hbm, emb_a_hbm, emb_t_hbm, fe_hbm,
                    idx_c_v, idx_a_v, idx_t_v, rows_c, rows_a, rows_t,
                    gsem, osem):
    wid = lax.axis_index("s") * _NC + lax.axis_index("c")
    base = H1 + wid * BPW2

    pltpu.sync_copy(idx_c_hbm.at[pl.ds(base, BPW2)], idx_c_v)
    pltpu.sync_copy(idx_a_hbm.at[pl.ds(base, BPW2)], idx_a_v)
    pltpu.sync_copy(idx_t_hbm.at[pl.ds(base, BPW2)], idx_t_v)

    c_c = pltpu.async_copy(emb_c_hbm.at[idx_c_v], rows_c, gsem)
    c_a = pltpu.async_copy(emb_a_hbm.at[idx_a_v], rows_a, gsem)
    c_t = pltpu.async_copy(emb_t_hbm.at[idx_t_v], rows_t, gsem)
    c_c.wait()
    c_a.wait()
    c_t.wait()

    out_base = wid * BPW2
    w_c = pltpu.async_copy(rows_c, fe_hbm.at[pl.ds(out_base, BPW2), pl.ds(0, D)], osem)
    w_a = pltpu.async_copy(rows_a, fe_hbm.at[pl.ds(out_base, BPW2), pl.ds(D, D)], osem)
    w_t = pltpu.async_copy(rows_t, fe_hbm.at[pl.ds(out_base, BPW2), pl.ds(2 * D, D)], osem)
    w_c.wait()
    w_a.wait()
    w_t.wait()


def _sc_gather(idx_C, idx_A, idx_T, emb_C, emb_A, emb_T):
    mesh = plsc.VectorSubcoreMesh(core_axis_name="c", subcore_axis_name="s",
                                  num_cores=_NC, num_subcores=_NS)
    run = functools.partial(
        pl.kernel,
        mesh=mesh,
        out_type=jax.ShapeDtypeStruct((H2, D3), jnp.float32),
        scratch_types=[
            pltpu.VMEM((BPW2,), jnp.int32),
            pltpu.VMEM((BPW2,), jnp.int32),
            pltpu.VMEM((BPW2,), jnp.int32),
            pltpu.VMEM((BPW2, D), jnp.float32),
            pltpu.VMEM((BPW2, D), jnp.float32),
            pltpu.VMEM((BPW2, D), jnp.float32),
            pltpu.SemaphoreType.DMA,
            pltpu.SemaphoreType.DMA,
        ],
    )(_sc_gather_body)
    return run(idx_C, idx_A, idx_T, emb_C, emb_A, emb_T)


def _tc1_kernel(idx_c_ref, idx_a_ref, idx_t_ref,
                emb_c_ref, emb_a_ref, emb_t_ref, out_ref):
    i = pl.program_id(0)
    b0 = i * BB
    for j in range(BB):
        c = idx_c_ref[b0 + j]
        a = idx_a_ref[b0 + j]
        t = idx_t_ref[b0 + j]
        row = jnp.concatenate([
            emb_c_ref[pl.ds(c, 1), :],
            emb_a_ref[pl.ds(a, 1), :],
            emb_t_ref[pl.ds(t, 1), :],
        ], axis=-1)
        out_ref[j, :, :] = jnp.broadcast_to(row, (N, D3))


def _tc_stage1(idx_C, idx_A, idx_T, emb_C, emb_A, emb_T):
    return pl.pallas_call(
        _tc1_kernel,
        grid_spec=pltpu.PrefetchScalarGridSpec(
            num_scalar_prefetch=3,
            grid=(H1 // BB,),
            in_specs=[
                pl.BlockSpec(emb_C.shape, lambda i, *_: (0, 0)),
                pl.BlockSpec(emb_A.shape, lambda i, *_: (0, 0)),
                pl.BlockSpec(emb_T.shape, lambda i, *_: (0, 0)),
            ],
            out_specs=pl.BlockSpec((BB, N, D3), lambda i, *_: (i, 0, 0)),
        ),
        out_shape=jax.ShapeDtypeStruct((B, N, D3), jnp.float32),
    )(idx_C, idx_A, idx_T, emb_C, emb_A, emb_T)


def _tc2_kernel(fe_ref, _aliased_ref, out_ref):
    fe = fe_ref[...].reshape(BB, 1, D3)
    out_ref[...] = jnp.broadcast_to(fe, (BB, N, D3))


def _tc_stage2(fold_emb2, out1):
    return pl.pallas_call(
        _tc2_kernel,
        grid=(H2 // BB,),
        in_specs=[
            pl.BlockSpec((BB, D3), lambda i: (i, 0)),
            pl.BlockSpec(memory_space=pltpu.MemorySpace.HBM),
        ],
        out_specs=pl.BlockSpec((BB, N, D3), lambda i: (i + H1 // BB, 0, 0)),
        out_shape=jax.ShapeDtypeStruct((B, N, D3), jnp.float32),
        input_output_aliases={1: 0},
    )(fold_emb2, out1)


def kernel(x_t, idx_C, idx_A, idx_T, emb_C, emb_A, emb_T):
    ic = idx_C.astype(jnp.int32)
    ia = idx_A.astype(jnp.int32)
    it = idx_T.astype(jnp.int32)
    out1 = _tc_stage1(ic, ia, it, emb_C, emb_A, emb_T)
    fe2 = _sc_gather(ic, ia, it, emb_C, emb_A, emb_T)
    return _tc_stage2(fe2, out1)
